# baseline (speedup 1.0000x reference)
import functools

import jax
import jax.numpy as jnp
from jax import lax
from jax.experimental import pallas as pl
from jax.experimental.pallas import tpu as pltpu
from jax.experimental.pallas import tpu_sc as plsc

NC, NS, L = 2, 16, 16
CH = 128


def _mm_body(x_ref, w_ref, b_ref, o_ref, *, relu):
    acc = jnp.dot(x_ref[...], w_ref[...], preferred_element_type=jnp.float32,
                  precision=lax.Precision.HIGHEST)
    if b_ref is not None:
        acc = acc + b_ref[...]
    if relu:
        acc = jnp.maximum(acc, 0.0)
    o_ref[...] = acc


def _mm(x, w, b=None, relu=False, block_m=256):
    M, K = x.shape
    N = w.shape[1]
    bm = min(block_m, M)
    grid = (pl.cdiv(M, bm),)
    in_specs = [
        pl.BlockSpec((bm, K), lambda i: (i, 0)),
        pl.BlockSpec((K, N), lambda i: (0, 0)),
    ]
    args = [x, w]
    if b is not None:
        in_specs.append(pl.BlockSpec((1, N), lambda i: (0, 0)))
        args.append(b.reshape(1, N))
        body = functools.partial(_mm_body, relu=relu)
    else:
        body = functools.partial(
            lambda x_ref, w_ref, o_ref, relu: _mm_body(x_ref, w_ref, None, o_ref, relu=relu),
            relu=relu)
    return pl.pallas_call(
        body,
        grid=grid,
        in_specs=in_specs,
        out_specs=pl.BlockSpec((bm, N), lambda i: (i, 0)),
        out_shape=jax.ShapeDtypeStruct((M, N), jnp.float32),
    )(*args)


def _mm_slabs(x, w, nslab, block_m=400):
    M, K = x.shape
    N = w.shape[1]
    ds = N // nslab

    def body(x_ref, w_ref, *o_refs):
        acc = jnp.dot(x_ref[...], w_ref[...], preferred_element_type=jnp.float32,
                      precision=lax.Precision.HIGHEST)
        for j, o in enumerate(o_refs):
            o[...] = acc[:, j * ds:(j + 1) * ds]

    bm = min(block_m, M)
    return pl.pallas_call(
        body,
        grid=(pl.cdiv(M, bm),),
        in_specs=[pl.BlockSpec((bm, K), lambda i: (i, 0)),
                  pl.BlockSpec((K, N), lambda i: (0, 0))],
        out_specs=[pl.BlockSpec((bm, ds), lambda i: (i, 0))] * nslab,
        out_shape=[jax.ShapeDtypeStruct((M, ds), jnp.float32)] * nslab,
    )(x, w)


@functools.lru_cache(maxsize=None)
def _make_edge_kernel(Nt, Nacc, D, Ep, E, mode, has_ew, layout, nslab):
    assert Ep % (NC * NS * CH) == 0 and Nacc % (NS * 8) == 0
    pergraph = layout == "pergraph"
    epc = Ep // NS if pergraph else Ep // (NC * NS)
    nchunks = epc // CH
    rpt = Nacc // NS
    zr = next(z for z in (32, 16, 8) if rpt % z == 0)
    nz = rpt // zr
    same_ab = mode == "gcn"
    mesh = plsc.VectorSubcoreMesh(core_axis_name="c", subcore_axis_name="s")

    def body(*refs):
        it = iter(refs)
        tables = [next(it) for _ in range(nslab)]
        a_hbm = next(it)
        b_hbm = None if same_ab else next(it)
        ew_hbm = next(it) if has_ew else None
        mu_hbm = next(it) if mode == "gat" else None
        src_hbm = next(it)
        dst_hbm = next(it)
        out_hbm = next(it)
        sidx = next(it)
        didx = next(it)
        ew_v = next(it) if has_ew else None
        w_v = next(it)
        rows = next(it)
        atab = next(it)
        btab = atab if same_ab else next(it)
        mu_v = next(it) if mode == "gat" else None
        zbuf = next(it)
        acc = next(it)
        sem = next(it)

        c = lax.axis_index("c")
        s = lax.axis_index("s")
        src_shift = c * (Nt // 2) if pergraph else 0
        e_base = (c * Ep + s * epc) if pergraph else ((c * NS + s) * epc)

        def zb(i, carry):
            for j in range(D // L):
                zbuf[i, pl.ds(j * L, L)] = jnp.zeros((L,), jnp.float32)
            return carry
        lax.fori_loop(0, zr, zb, 0)
        if pergraph:
            pltpu.sync_copy(a_hbm.at[pl.ds(src_shift, Nt // 2)], atab)
            if not same_ab:
                pltpu.sync_copy(b_hbm.at[pl.ds(src_shift, Nt // 2)], btab)
        else:
            pltpu.sync_copy(a_hbm, atab)
            if not same_ab:
                pltpu.sync_copy(b_hbm, btab)
        if mode == "gat":
            pltpu.sync_copy(mu_hbm, mu_v)
        mu16 = mu_v[pl.ds(0, L)] if mode == "gat" else None

        for slab in range(nslab):
            h_hbm = tables[slab]
            for k in range(nz):
                pltpu.sync_copy(zbuf, acc.at[pl.ds(s * rpt + k * zr, zr)])
            plsc.subcore_barrier()

            def chunk(k, carry):
                off = e_base + k * CH
                pltpu.sync_copy(src_hbm.at[pl.ds(off, CH)], sidx)
                pltpu.sync_copy(dst_hbm.at[pl.ds(off, CH)], didx)
                if has_ew:
                    pltpu.sync_copy(ew_hbm.at[pl.ds(off, CH)], ew_v)
                pltpu.async_copy(h_hbm.at[sidx], rows, sem).wait()
                for g in range(CH // L):
                    sv = sidx[pl.ds(g * L, L)] - src_shift
                    dv = didx[pl.ds(g * L, L)]
                    a16 = plsc.load_gather(atab, [sv])
                    b16 = plsc.load_gather(btab, [dv])
                    if mode == "gat":
                        x = a16 + b16
                        x = jnp.where(x > 0, x, 0.2 * x)
                        w16 = jnp.exp(x - mu16)
                    else:
                        w16 = a16 * b16
                        if has_ew:
                            w16 = w16 * ew_v[pl.ds(g * L, L)]
                    pos0 = off - c * Ep if pergraph else off
                    eid = lax.iota(jnp.int32, L) + (pos0 + g * L)
                    w16 = jnp.where(eid < E, w16, 0.0)
                    w_v[pl.ds(g * L, L)] = w16

                def scale(e, carry2):
                    wv = plsc.load_gather(w_v, [jnp.zeros((L,), jnp.int32) + e])
                    for j in range(D // L):
                        rows[e, pl.ds(j * L, L)] = rows[e, pl.ds(j * L, L)] * wv
                    return carry2
                lax.fori_loop(0, CH, scale, 0)
                pltpu.sync_copy(rows, acc.at[didx], add=True)
                return carry
            lax.fori_loop(0, nchunks, chunk, 0)
            plsc.subcore_barrier()
            pltpu.sync_copy(acc.at[pl.ds(s * rpt, rpt)],
                            out_hbm.at[slab, c, pl.ds(s * rpt, rpt)])

    scratch = [pltpu.VMEM((CH,), jnp.int32), pltpu.VMEM((CH,), jnp.int32)]
    if has_ew:
        scratch.append(pltpu.VMEM((CH,), jnp.float32))
    ntab = Nt // 2 if pergraph else Nt
    scratch += [pltpu.VMEM((CH,), jnp.float32), pltpu.VMEM((CH, D), jnp.float32),
                pltpu.VMEM((ntab,), jnp.float32)]
    if not same_ab:
        scratch.append(pltpu.VMEM((ntab,), jnp.float32))
    if mode == "gat":
        scratch.append(pltpu.VMEM((L,), jnp.float32))
    scratch += [pltpu.VMEM((zr, D), jnp.float32),
                pltpu.VMEM_SHARED((Nacc, D), jnp.float32),
                pltpu.SemaphoreType.DMA]

    return pl.kernel(
        body,
        out_type=jax.ShapeDtypeStruct((nslab, NC, Nacc, D), jnp.float32),
        mesh=mesh,
        scratch_types=scratch,
        compiler_params=pltpu.CompilerParams(needs_layout_passes=False),
    )


def _pad_edges(src, dst, ew=None):
    E = src.shape[0]
    Ep = pl.cdiv(E, NC * NS * CH) * NC * NS * CH
    src = jnp.pad(src.astype(jnp.int32), (0, Ep - E))
    dst = jnp.pad(dst.astype(jnp.int32), (0, Ep - E))
    if ew is not None:
        ew = jnp.pad(ew, (0, Ep - E))
    return src, dst, ew, Ep, E


def _gcn_finalize(parts, gi, h, dis, b, base, n, block_m=400):
    D = h.shape[1]
    bm = block_m
    bb = base // bm
    dis2 = (dis * dis).reshape(-1, 1)

    def body(p_ref, h_ref, d_ref, b_ref, o_ref):
        agg = p_ref[0, 0] + p_ref[0, 1] if gi is None else p_ref[0, 0]
        v = agg + d_ref[...] * h_ref[...] + b_ref[...]
        o_ref[...] = jnp.maximum(v, 0.0)

    pspec = (pl.BlockSpec((1, 2, bm, D), lambda i: (0, 0, i, 0)) if gi is None
             else pl.BlockSpec((1, 1, bm, D), lambda i: (0, gi, i, 0)))
    return pl.pallas_call(
        body,
        grid=(n // bm,),
        in_specs=[
            pspec,
            pl.BlockSpec((bm, D), lambda i: (bb + i, 0)),
            pl.BlockSpec((bm, 1), lambda i: (bb + i, 0)),
            pl.BlockSpec((1, D), lambda i: (0, 0)),
        ],
        out_specs=pl.BlockSpec((bm, D), lambda i: (i, 0)),
        out_shape=jax.ShapeDtypeStruct((n, D), jnp.float32),
    )(parts, h, dis2, b.reshape(1, D))


def _gat_finalize(parts, gi, hw80, asrc, adst, mu, b, base, n, block_m=400):
    bm = block_m
    bb = base // bm

    def body(p_ref, h_ref, a_ref, d_ref, m_ref, b_ref, o_ref):
        x = a_ref[...] + d_ref[...]
        exs = jnp.exp(jnp.where(x > 0, x, 0.2 * x) - m_ref[0, 0])
        hw = h_ref[...]
        agg = p_ref[0, 0]
        num = agg[:, 0:64] + exs * hw[:, 0:64]
        den = agg[:, 64:65] + exs
        v = num / jnp.maximum(den, 1e-16) + b_ref[...]
        o_ref[...] = jnp.maximum(v, 0.0)

    return pl.pallas_call(
        body,
        grid=(n // bm,),
        in_specs=[
            pl.BlockSpec((1, 1, bm, 128), lambda i: (0, gi, i, 0)),
            pl.BlockSpec((bm, 128), lambda i: (bb + i, 0)),
            pl.BlockSpec((bm, 1), lambda i: (bb + i, 0)),
            pl.BlockSpec((bm, 1), lambda i: (bb + i, 0)),
            pl.BlockSpec((1, 1), lambda i: (0, 0)),
            pl.BlockSpec((1, 64), lambda i: (0, 0)),
        ],
        out_specs=pl.BlockSpec((bm, 64), lambda i: (i, 0)),
        out_shape=jax.ShapeDtypeStruct((n, 64), jnp.float32),
    )(parts, hw80, asrc, adst, mu.reshape(1, 1), b.reshape(1, 64))


def _ami_finalize(parts8, t8, dis, b1024, li, n, block_m=400):
    bm = block_m
    dis2 = (dis * dis).reshape(-1, 1)

    def body(*refs):
        p_ref = refs[0]
        t = refs[1:9]
        d_ref, b_ref, li_ref = refs[9], refs[10], refs[11]
        o_ref = refs[12]
        for j in range(8):
            v = (p_ref[j, 0] + p_ref[j, 1] + d_ref[...] * t[j][...]
                 + b_ref[:, j * 128:(j + 1) * 128])
            v = jnp.maximum(v, 0.0)
            if j == 7:
                cols = lax.broadcasted_iota(jnp.int32, v.shape, 1)
                v = jnp.where(cols == 127, li_ref[...], v)
            o_ref[:, j * 128:(j + 1) * 128] = v

    specs = ([pl.BlockSpec((8, 2, bm, 128), lambda i: (0, 0, i, 0))]
             + [pl.BlockSpec((bm, 128), lambda i: (i, 0))] * 8
             + [pl.BlockSpec((bm, 1), lambda i: (i, 0)),
                pl.BlockSpec((1, 1024), lambda i: (0, 0)),
                pl.BlockSpec((bm, 1), lambda i: (i, 0))])
    return pl.pallas_call(
        body,
        grid=(n // bm,),
        in_specs=specs,
        out_specs=pl.BlockSpec((bm, 1024), lambda i: (i, 0)),
        out_shape=jax.ShapeDtypeStruct((n, 1024), jnp.float32),
    )(parts8, *t8, dis2, b1024.reshape(1, 1024), li.reshape(-1, 1))


def _pool_body(b_ref, x_ref, o_ref, acc, cnt, *, nseg):
    i = pl.program_id(0)

    @pl.when(i == 0)
    def _():
        acc[...] = jnp.zeros_like(acc)
        cnt[...] = jnp.zeros_like(cnt)

    bvals = b_ref[0, 0, :]
    oh = (bvals[:, None] == lax.broadcasted_iota(jnp.int32, (bvals.shape[0], nseg), 1)
          ).astype(jnp.float32)
    acc[...] += lax.dot_general(oh, x_ref[...], (((0,), (0,)), ((), ())),
                                preferred_element_type=jnp.float32,
                                precision=lax.Precision.HIGHEST)
    cnt[...] += jnp.sum(oh, axis=0, keepdims=True)

    @pl.when(i == pl.num_programs(0) - 1)
    def _():
        o_ref[...] = acc[...] / jnp.maximum(cnt[...], 1.0).T


def _pool(x, batch, nseg, block_n=256):
    n, d = x.shape
    npad = pl.cdiv(n, block_n) * block_n
    if npad != n:
        x = jnp.pad(x, ((0, npad - n), (0, 0)))
        batch = jnp.pad(batch, (0, npad - n), constant_values=nseg)
    b3 = batch.reshape(npad // block_n, 1, block_n).astype(jnp.int32)
    return pl.pallas_call(
        functools.partial(_pool_body, nseg=nseg),
        grid=(npad // block_n,),
        in_specs=[
            pl.BlockSpec((1, 1, block_n), lambda i: (i, 0, 0)),
            pl.BlockSpec((block_n, d), lambda i: (i, 0)),
        ],
        out_specs=pl.BlockSpec((nseg, d), lambda i: (0, 0)),
        out_shape=jax.ShapeDtypeStruct((nseg, d), jnp.float32),
        scratch_shapes=[pltpu.VMEM((nseg, d), jnp.float32),
                        pltpu.VMEM((1, nseg), jnp.float32)],
    )(b3, x)


def _head_body(p1, p2, p3, w1a, w1b, w1c, b1, w2, b2, w3, b3, o_ref):
    dot = functools.partial(jnp.dot, preferred_element_type=jnp.float32,
                            precision=lax.Precision.HIGHEST)
    z = (dot(p1[...], w1a[...]) + dot(p2[...], w1b[...])
         + dot(p3[...], w1c[...]) + b1[...])
    z = jnp.maximum(z, 0.0)
    z = jnp.maximum(dot(z, w2[...]) + b2[...], 0.0)
    o_ref[...] = dot(z, w3[...]) + b3[...]


def _head(p1, p2, p3, fc1_W, fc1_b, fc2_W, fc2_b, out_W, out_b):
    B = p1.shape[0]
    full = lambda s: pl.BlockSpec(s, lambda: tuple(0 for _ in s))
    args = [p1, p2, p3, fc1_W[0:128], fc1_W[128:256], fc1_W[256:384],
            fc1_b.reshape(1, -1), fc2_W, fc2_b.reshape(1, -1),
            out_W, out_b.reshape(1, -1)]
    return pl.pallas_call(
        _head_body,
        in_specs=[full(a.shape) for a in args],
        out_specs=full((B, 1)),
        out_shape=jax.ShapeDtypeStruct((B, 1), jnp.float32),
    )(*args)


def kernel(x1, edge_index1, edge_attr1, batch1, x2, edge_index2, edge_attr2,
           batch2, x3, edge_index3, ami_dis, ami_dis_li, drug_feature, batch3,
           w1, b1, w2, b2, gat_W, gat_att_src, gat_att_dst, gat_b,
           g2_W, g2_b, g3_W, g3_b, g4_W, g4_b,
           fc1_W, fc1_b, fc2_W, fc2_b, out_W, out_b):
    n1 = x1.shape[0]
    n2 = x2.shape[0]
    n12 = n1 + n2
    B = drug_feature.shape[0]
    assert n1 == n2

    x = jnp.concatenate([x1, x2], axis=0)
    h = _mm(x, w1, b1, relu=True)
    h = _mm(h, w2, b2, relu=True)

    s1, d1, _, Ep1, E1 = _pad_edges(edge_index1[0], edge_index1[1])
    s2, d2, _, Ep2, E2 = _pad_edges(edge_index2[0] + n1, edge_index2[1])
    assert Ep1 == Ep2 and E1 == E2
    scat = jnp.concatenate([s1, s2])
    dcat = jnp.concatenate([d1, d2])
    nap = pl.cdiv(n1, 1024) * 1024

    W128 = jnp.pad(gat_W, ((0, 0), (0, 64)))
    b128 = jnp.zeros((128,), jnp.float32).at[64].set(1.0)
    hw80 = _mm(h, W128, b128)
    asrc = _mm(hw80, jnp.pad(gat_att_src, (0, 64)).reshape(-1, 1))
    adst = _mm(hw80, jnp.pad(gat_att_dst, (0, 64)).reshape(-1, 1))
    mu = jax.nn.leaky_relu(jnp.max(asrc) + jnp.max(adst), 0.2)
    mu16 = jnp.full((L,), mu, jnp.float32)

    gat_k = _make_edge_kernel(n12, nap, 128, Ep1, E1, "gat", False,
                              "pergraph", 1)
    pa = gat_k(hw80, asrc[:, 0], adst[:, 0], mu16, scat, dcat)
    g1 = _gat_finalize(pa, 0, hw80, asrc, adst, mu, gat_b, 0, n1)
    g2 = _gat_finalize(pa, 1, hw80, asrc, adst, mu, gat_b, n1, n2)
    h = jnp.concatenate([g1, g2], axis=0)

    hw = _mm(h, g2_W)
    deg = jax.ops.segment_sum(jnp.ones(E1 + E2, jnp.float32),
                              jnp.concatenate([edge_index1[1],
                                               edge_index2[1] + n1]),
                              num_segments=n12) + 1.0
    dis = lax.rsqrt(deg)
    gcn_k = _make_edge_kernel(n12, nap, 128, Ep1, E1, "gcn", False,
                              "pergraph", 1)
    pc = gcn_k(hw, dis, scat, dcat)
    c1 = _gcn_finalize(pc, 0, hw, dis, g2_b, 0, n1)
    c2 = _gcn_finalize(pc, 1, hw, dis, g2_b, n1, n2)

    bcat = jnp.concatenate([batch1.astype(jnp.int32),
                            batch2.astype(jnp.int32) + B])
    p12 = _pool(jnp.concatenate([c1, c2], axis=0), bcat, 2 * B)
    p1, p2 = p12[:B], p12[B:]

    n3 = x3.shape[0]
    nap3 = pl.cdiv(n3, 1024) * 1024
    s3, d3, ew3, Ep3, E3 = _pad_edges(edge_index3[0], edge_index3[1], ami_dis)
    ew3 = ew3 + 0.0 * c2[0, 0]
    deg3 = jax.ops.segment_sum(ami_dis, edge_index3[1], num_segments=n3) + 1.0
    dis3 = lax.rsqrt(deg3)

    g3_Wp = jnp.pad(g3_W, ((0, 0), (0, 1)))
    t8 = _mm_slabs(x3, g3_Wp, 8)
    slab_k = _make_edge_kernel(n3, nap3, 128, Ep3, E3, "gcn", True,
                               "split", 8)
    parts8 = slab_k(*t8, dis3, ew3, s3, d3)
    ami = _ami_finalize(parts8, t8, dis3, jnp.pad(g3_b, (0, 1)), ami_dis_li, n3)

    h4 = _mm(ami, g4_W)
    g4_k = _make_edge_kernel(n3, nap3, 128, Ep3, E3, "gcn", True,
                             "split", 1)
    p4 = g4_k(h4, dis3, ew3, s3, d3)
    h4f = _gcn_finalize(p4, None, h4, dis3, g4_b, 0, n3)
    p3 = _pool(h4f, batch3.astype(jnp.int32), B)

    return _head(p1, p2, p3, fc1_W, fc1_b, fc2_W, fc2_b, out_W, out_b)


# speedup vs baseline: 1.3795x; 1.3795x over previous
import functools

import jax
import jax.numpy as jnp
from jax import lax
from jax.experimental import pallas as pl
from jax.experimental.pallas import tpu as pltpu
from jax.experimental.pallas import tpu_sc as plsc

NC, NS, L = 2, 16, 16
CH = 128


def _mm_body(x_ref, w_ref, b_ref, o_ref, *, relu):
    acc = jnp.dot(x_ref[...], w_ref[...], preferred_element_type=jnp.float32,
                  precision=lax.Precision.HIGHEST)
    if b_ref is not None:
        acc = acc + b_ref[...]
    if relu:
        acc = jnp.maximum(acc, 0.0)
    o_ref[...] = acc


def _mm(x, w, b=None, relu=False, block_m=256):
    M, K = x.shape
    N = w.shape[1]
    bm = min(block_m, M)
    grid = (pl.cdiv(M, bm),)
    in_specs = [
        pl.BlockSpec((bm, K), lambda i: (i, 0)),
        pl.BlockSpec((K, N), lambda i: (0, 0)),
    ]
    args = [x, w]
    if b is not None:
        in_specs.append(pl.BlockSpec((1, N), lambda i: (0, 0)))
        args.append(b.reshape(1, N))
        body = functools.partial(_mm_body, relu=relu)
    else:
        body = functools.partial(
            lambda x_ref, w_ref, o_ref, relu: _mm_body(x_ref, w_ref, None, o_ref, relu=relu),
            relu=relu)
    return pl.pallas_call(
        body,
        grid=grid,
        in_specs=in_specs,
        out_specs=pl.BlockSpec((bm, N), lambda i: (i, 0)),
        out_shape=jax.ShapeDtypeStruct((M, N), jnp.float32),
    )(*args)


def _mm_slabs(x, w, nslab, block_m=400):
    M, K = x.shape
    N = w.shape[1]
    ds = N // nslab

    def body(x_ref, w_ref, *o_refs):
        acc = jnp.dot(x_ref[...], w_ref[...], preferred_element_type=jnp.float32,
                      precision=lax.Precision.HIGHEST)
        for j, o in enumerate(o_refs):
            o[...] = acc[:, j * ds:(j + 1) * ds]

    bm = min(block_m, M)
    return pl.pallas_call(
        body,
        grid=(pl.cdiv(M, bm),),
        in_specs=[pl.BlockSpec((bm, K), lambda i: (i, 0)),
                  pl.BlockSpec((K, N), lambda i: (0, 0))],
        out_specs=[pl.BlockSpec((bm, ds), lambda i: (i, 0))] * nslab,
        out_shape=[jax.ShapeDtypeStruct((M, ds), jnp.float32)] * nslab,
    )(x, w)


NB = 8


@functools.lru_cache(maxsize=None)
def _make_edge_kernel(Nt, Nacc, D, Et, ept, mode, has_ew, layout, nslab):
    assert ept % (CH * NB) == 0 and Nacc % (NS * 8) == 0
    pergraph = layout == "pergraph"
    nchunks = pl.cdiv(Et, CH)
    nblocks = pl.cdiv(nchunks, NB)
    rstride = ept // CH
    rpt = Nacc // NS
    same_ab = mode == "gcn"
    ntab = Nt // 2 if pergraph else Nt
    mesh = plsc.VectorSubcoreMesh(core_axis_name="c", subcore_axis_name="s")

    def body(*refs):
        it = iter(refs)
        tables = [next(it) for _ in range(nslab)]
        ab_hbm = next(it)
        ew_hbm = next(it) if has_ew else None
        mu_hbm = next(it) if mode == "gat" else None
        s2_hbm = next(it)
        d2_hbm = next(it)
        z_hbm = next(it)
        out_hbm = next(it)
        sblk = next(it)
        dblk = next(it)
        ewblk = next(it) if has_ew else None
        w_v = next(it)
        rows2 = next(it)
        abtab = next(it)
        mu_v = next(it) if mode == "gat" else None
        acc = next(it)
        semg = next(it)

        c = lax.axis_index("c")
        s = lax.axis_index("s")
        src_shift = c * (Nt // 2) if pergraph else 0
        r_base = (c * NS + s) * rstride

        pltpu.sync_copy(z_hbm, acc.at[pl.ds(s * rpt, rpt)])
        if pergraph:
            pltpu.sync_copy(ab_hbm.at[pl.ds(src_shift, ntab)], abtab)
        else:
            pltpu.sync_copy(ab_hbm, abtab)
        if mode == "gat":
            pltpu.sync_copy(mu_hbm, mu_v)
        plsc.subcore_barrier()
        mu16 = mu_v[pl.ds(0, L)] if mode == "gat" else None

        def stage(b, buf):
            pltpu.sync_copy(s2_hbm.at[pl.ds(r_base + b * NB, NB)], sblk.at[buf])
            pltpu.sync_copy(d2_hbm.at[pl.ds(r_base + b * NB, NB)], dblk.at[buf])
            if has_ew:
                pltpu.sync_copy(ew_hbm.at[pl.ds(r_base + b * NB, NB)],
                                ewblk.at[buf])

        for slab in range(nslab):
            h_hbm = tables[slab]
            if slab > 0:
                pltpu.sync_copy(z_hbm, acc.at[pl.ds(s * rpt, rpt)])
                plsc.subcore_barrier()
            stage(0, 0)
            pltpu.async_copy(h_hbm.at[sblk.at[0, 0]], rows2.at[0], semg)

            def chunk(j, carry):
                p = j % 2
                q = (j // NB) % 2
                jb = j % NB
                pltpu.make_async_copy(h_hbm.at[pl.ds(0, CH)], rows2.at[p],
                                      semg).wait()

                @pl.when(j + 1 < nchunks)
                def _():
                    j1 = j + 1
                    pltpu.async_copy(
                        h_hbm.at[sblk.at[(j1 // NB) % 2, j1 % NB]],
                        rows2.at[1 - p], semg)

                for g in range(CH // L):
                    dv = dblk[q, jb, pl.ds(g * L, L)]
                    b16 = plsc.load_gather(abtab, [dv])
                    if mode == "gat":
                        a16 = plsc.load_gather(
                            rows2, [jnp.zeros((L,), jnp.int32) + p,
                                    lax.iota(jnp.int32, L) + g * L,
                                    jnp.zeros((L,), jnp.int32) + 65])
                        x = a16 + b16
                        x = jnp.where(x > 0, x, 0.2 * x)
                        w16 = jnp.exp(x - mu16)
                    else:
                        sv = sblk[q, jb, pl.ds(g * L, L)] - src_shift
                        a16 = plsc.load_gather(abtab, [sv])
                        w16 = a16 * b16
                        if has_ew:
                            w16 = w16 * ewblk[q, jb, pl.ds(g * L, L)]
                    eid = lax.iota(jnp.int32, L) + (j * CH + g * L)
                    w16 = jnp.where(eid < Et, w16, 0.0)
                    w_v[pl.ds(g * L, L)] = w16

                def scale(e, carry2):
                    wv = plsc.load_gather(w_v, [jnp.zeros((L,), jnp.int32) + e])
                    for jj in range(D // L):
                        rows2[p, e, pl.ds(jj * L, L)] = (
                            rows2[p, e, pl.ds(jj * L, L)] * wv)
                    return carry2
                lax.fori_loop(0, CH, scale, 0)

                pltpu.sync_copy(rows2.at[p], acc.at[dblk.at[q, jb]], add=True)

                @pl.when((jb == 0) & (j // NB + 1 < nblocks))
                def _():
                    stage(j // NB + 1, 1 - q)
                return carry
            lax.fori_loop(0, nchunks, chunk, 0)
            plsc.subcore_barrier()
            pltpu.sync_copy(acc.at[pl.ds(s * rpt, rpt)],
                            out_hbm.at[slab, c, pl.ds(s * rpt, rpt)])

    scratch = [pltpu.VMEM((2, NB, CH), jnp.int32),
               pltpu.VMEM((2, NB, CH), jnp.int32)]
    if has_ew:
        scratch.append(pltpu.VMEM((2, NB, CH), jnp.float32))
    scratch += [pltpu.VMEM((CH,), jnp.float32),
                pltpu.VMEM((2, CH, D), jnp.float32),
                pltpu.VMEM((ntab,), jnp.float32)]
    if mode == "gat":
        scratch.append(pltpu.VMEM((L,), jnp.float32))
    scratch += [pltpu.VMEM_SHARED((Nacc, D), jnp.float32),
                pltpu.SemaphoreType.DMA]

    return pl.kernel(
        body,
        out_type=jax.ShapeDtypeStruct((nslab, NC, Nacc, D), jnp.float32),
        mesh=mesh,
        scratch_types=scratch,
        compiler_params=pltpu.CompilerParams(needs_layout_passes=False),
    )


def _tile_edges(x, ntile, ept):
    E = x.shape[0]
    assert E % ntile == 0
    seg = jnp.pad(x.reshape(ntile, E // ntile), ((0, 0), (0, ept - E // ntile)))
    flat = seg.reshape(-1)
    n = flat.shape[0]
    return jnp.pad(flat, (0, (NB - 1) * CH)).reshape(n // CH + NB - 1, CH)


def _gcn_finalize(parts, gi, h, dis, b, base, n, block_m=400):
    D = h.shape[1]
    bm = block_m
    bb = base // bm
    dis2 = (dis * dis).reshape(-1, 1)

    def body(p_ref, h_ref, d_ref, b_ref, o_ref):
        agg = p_ref[0, 0] + p_ref[0, 1] if gi is None else p_ref[0, 0]
        v = agg + d_ref[...] * h_ref[...] + b_ref[...]
        o_ref[...] = jnp.maximum(v, 0.0)

    pspec = (pl.BlockSpec((1, 2, bm, D), lambda i: (0, 0, i, 0)) if gi is None
             else pl.BlockSpec((1, 1, bm, D), lambda i: (0, gi, i, 0)))
    return pl.pallas_call(
        body,
        grid=(n // bm,),
        in_specs=[
            pspec,
            pl.BlockSpec((bm, D), lambda i: (bb + i, 0)),
            pl.BlockSpec((bm, 1), lambda i: (bb + i, 0)),
            pl.BlockSpec((1, D), lambda i: (0, 0)),
        ],
        out_specs=pl.BlockSpec((bm, D), lambda i: (i, 0)),
        out_shape=jax.ShapeDtypeStruct((n, D), jnp.float32),
    )(parts, h, dis2, b.reshape(1, D))


def _gat_finalize(parts, gi, hw80, mu, b, base, n, block_m=400):
    bm = block_m
    bb = base // bm

    def body(p_ref, h_ref, m_ref, b_ref, o_ref):
        hw = h_ref[...]
        x = hw[:, 65:66] + hw[:, 66:67]
        exs = jnp.exp(jnp.where(x > 0, x, 0.2 * x) - m_ref[0, 0])
        agg = p_ref[0, 0]
        num = agg[:, 0:64] + exs * hw[:, 0:64]
        den = agg[:, 64:65] + exs
        v = num / jnp.maximum(den, 1e-16) + b_ref[...]
        o_ref[...] = jnp.maximum(v, 0.0)

    return pl.pallas_call(
        body,
        grid=(n // bm,),
        in_specs=[
            pl.BlockSpec((1, 1, bm, 128), lambda i: (0, gi, i, 0)),
            pl.BlockSpec((bm, 128), lambda i: (bb + i, 0)),
            pl.BlockSpec((1, 1), lambda i: (0, 0)),
            pl.BlockSpec((1, 64), lambda i: (0, 0)),
        ],
        out_specs=pl.BlockSpec((bm, 64), lambda i: (i, 0)),
        out_shape=jax.ShapeDtypeStruct((n, 64), jnp.float32),
    )(parts, hw80, mu.reshape(1, 1), b.reshape(1, 64))


def _ami_finalize(parts8, t8, dis, b1024, li, n, block_m=400):
    bm = block_m
    dis2 = (dis * dis).reshape(-1, 1)

    def body(*refs):
        p_ref = refs[0]
        t = refs[1:9]
        d_ref, b_ref, li_ref = refs[9], refs[10], refs[11]
        o_ref = refs[12]
        for j in range(8):
            v = (p_ref[j, 0] + p_ref[j, 1] + d_ref[...] * t[j][...]
                 + b_ref[:, j * 128:(j + 1) * 128])
            v = jnp.maximum(v, 0.0)
            if j == 7:
                cols = lax.broadcasted_iota(jnp.int32, v.shape, 1)
                v = jnp.where(cols == 127, li_ref[...], v)
            o_ref[:, j * 128:(j + 1) * 128] = v

    specs = ([pl.BlockSpec((8, 2, bm, 128), lambda i: (0, 0, i, 0))]
             + [pl.BlockSpec((bm, 128), lambda i: (i, 0))] * 8
             + [pl.BlockSpec((bm, 1), lambda i: (i, 0)),
                pl.BlockSpec((1, 1024), lambda i: (0, 0)),
                pl.BlockSpec((bm, 1), lambda i: (i, 0))])
    return pl.pallas_call(
        body,
        grid=(n // bm,),
        in_specs=specs,
        out_specs=pl.BlockSpec((bm, 1024), lambda i: (i, 0)),
        out_shape=jax.ShapeDtypeStruct((n, 1024), jnp.float32),
    )(parts8, *t8, dis2, b1024.reshape(1, 1024), li.reshape(-1, 1))


def _pool_body(b_ref, x_ref, o_ref, acc, cnt, *, nseg):
    i = pl.program_id(0)

    @pl.when(i == 0)
    def _():
        acc[...] = jnp.zeros_like(acc)
        cnt[...] = jnp.zeros_like(cnt)

    bvals = b_ref[0, 0, :]
    oh = (bvals[:, None] == lax.broadcasted_iota(jnp.int32, (bvals.shape[0], nseg), 1)
          ).astype(jnp.float32)
    acc[...] += lax.dot_general(oh, x_ref[...], (((0,), (0,)), ((), ())),
                                preferred_element_type=jnp.float32,
                                precision=lax.Precision.HIGHEST)
    cnt[...] += jnp.sum(oh, axis=0, keepdims=True)

    @pl.when(i == pl.num_programs(0) - 1)
    def _():
        o_ref[...] = acc[...] / jnp.maximum(cnt[...], 1.0).T


def _pool(x, batch, nseg, block_n=256):
    n, d = x.shape
    npad = pl.cdiv(n, block_n) * block_n
    if npad != n:
        x = jnp.pad(x, ((0, npad - n), (0, 0)))
        batch = jnp.pad(batch, (0, npad - n), constant_values=nseg)
    b3 = batch.reshape(npad // block_n, 1, block_n).astype(jnp.int32)
    return pl.pallas_call(
        functools.partial(_pool_body, nseg=nseg),
        grid=(npad // block_n,),
        in_specs=[
            pl.BlockSpec((1, 1, block_n), lambda i: (i, 0, 0)),
            pl.BlockSpec((block_n, d), lambda i: (i, 0)),
        ],
        out_specs=pl.BlockSpec((nseg, d), lambda i: (0, 0)),
        out_shape=jax.ShapeDtypeStruct((nseg, d), jnp.float32),
        scratch_shapes=[pltpu.VMEM((nseg, d), jnp.float32),
                        pltpu.VMEM((1, nseg), jnp.float32)],
    )(b3, x)


def _head_body(p1, p2, p3, w1a, w1b, w1c, b1, w2, b2, w3, b3, o_ref):
    dot = functools.partial(jnp.dot, preferred_element_type=jnp.float32,
                            precision=lax.Precision.HIGHEST)
    z = (dot(p1[...], w1a[...]) + dot(p2[...], w1b[...])
         + dot(p3[...], w1c[...]) + b1[...])
    z = jnp.maximum(z, 0.0)
    z = jnp.maximum(dot(z, w2[...]) + b2[...], 0.0)
    o_ref[...] = dot(z, w3[...]) + b3[...]


def _head(p1, p2, p3, fc1_W, fc1_b, fc2_W, fc2_b, out_W, out_b):
    B = p1.shape[0]
    full = lambda s: pl.BlockSpec(s, lambda: tuple(0 for _ in s))
    args = [p1, p2, p3, fc1_W[0:128], fc1_W[128:256], fc1_W[256:384],
            fc1_b.reshape(1, -1), fc2_W, fc2_b.reshape(1, -1),
            out_W, out_b.reshape(1, -1)]
    return pl.pallas_call(
        _head_body,
        in_specs=[full(a.shape) for a in args],
        out_specs=full((B, 1)),
        out_shape=jax.ShapeDtypeStruct((B, 1), jnp.float32),
    )(*args)


def kernel(x1, edge_index1, edge_attr1, batch1, x2, edge_index2, edge_attr2,
           batch2, x3, edge_index3, ami_dis, ami_dis_li, drug_feature, batch3,
           w1, b1, w2, b2, gat_W, gat_att_src, gat_att_dst, gat_b,
           g2_W, g2_b, g3_W, g3_b, g4_W, g4_b,
           fc1_W, fc1_b, fc2_W, fc2_b, out_W, out_b):
    n1 = x1.shape[0]
    n2 = x2.shape[0]
    n12 = n1 + n2
    B = drug_feature.shape[0]
    assert n1 == n2
    unit = NC * NS * CH

    x = jnp.concatenate([x1, x2], axis=0)
    h = _mm(x, w1, b1, relu=True)
    h = _mm(h, w2, b2, relu=True)

    E1 = edge_index1.shape[1]
    E2 = edge_index2.shape[1]
    assert E1 == E2
    Et12 = E1 // NS
    ept12 = pl.cdiv(Et12, CH * NB) * CH * NB
    scat2d = _tile_edges(jnp.concatenate([
        edge_index1[0].astype(jnp.int32),
        edge_index2[0].astype(jnp.int32) + n1]), NC * NS, ept12)
    dcat2d = _tile_edges(jnp.concatenate([
        edge_index1[1].astype(jnp.int32),
        edge_index2[1].astype(jnp.int32)]), NC * NS, ept12)
    nap = pl.cdiv(n1, NS * 8) * NS * 8
    zblk = jnp.zeros((nap // NS, 128), jnp.float32)

    W128 = jnp.pad(gat_W, ((0, 0), (0, 64)))
    W128 = W128.at[:, 65].set(gat_W @ gat_att_src)
    W128 = W128.at[:, 66].set(gat_W @ gat_att_dst)
    b128 = jnp.zeros((128,), jnp.float32).at[64].set(1.0)
    hw80 = _mm(h, W128, b128)
    adst1d = hw80[:, 66]
    mu = jax.nn.leaky_relu(jnp.max(hw80[:, 65]) + jnp.max(adst1d), 0.2)
    mu16 = jnp.full((L,), mu, jnp.float32)

    gat_k = _make_edge_kernel(n12, nap, 128, Et12, ept12, "gat", False,
                              "pergraph", 1)
    pa = gat_k(hw80, adst1d, mu16, scat2d, dcat2d, zblk)
    g1 = _gat_finalize(pa, 0, hw80, mu, gat_b, 0, n1)
    g2 = _gat_finalize(pa, 1, hw80, mu, gat_b, n1, n2)
    h = jnp.concatenate([g1, g2], axis=0)

    hw = _mm(h, g2_W)
    deg = jax.ops.segment_sum(jnp.ones(E1 + E2, jnp.float32),
                              jnp.concatenate([edge_index1[1],
                                               edge_index2[1] + n1]),
                              num_segments=n12) + 1.0
    dis = lax.rsqrt(deg)
    gcn_k = _make_edge_kernel(n12, nap, 128, Et12, ept12, "gcn", False,
                              "pergraph", 1)
    pc = gcn_k(hw, dis, scat2d, dcat2d, zblk)
    c1 = _gcn_finalize(pc, 0, hw, dis, g2_b, 0, n1)
    c2 = _gcn_finalize(pc, 1, hw, dis, g2_b, n1, n2)

    bcat = jnp.concatenate([batch1.astype(jnp.int32),
                            batch2.astype(jnp.int32) + B])
    p12 = _pool(jnp.concatenate([c1, c2], axis=0), bcat, 2 * B)
    p1, p2 = p12[:B], p12[B:]

    n3 = x3.shape[0]
    nap3 = pl.cdiv(n3, NS * 8) * NS * 8
    E3 = edge_index3.shape[1]
    Et3 = E3 // (NC * NS)
    ept3 = pl.cdiv(Et3, CH * NB) * CH * NB
    s3_2d = _tile_edges(edge_index3[0].astype(jnp.int32), NC * NS, ept3)
    d3_2d = _tile_edges(edge_index3[1].astype(jnp.int32), NC * NS, ept3)
    deg3 = jax.ops.segment_sum(ami_dis, edge_index3[1], num_segments=n3) + 1.0
    dis3 = lax.rsqrt(deg3)

    g3_Wp = jnp.pad(g3_W, ((0, 0), (0, 1)))
    t8 = _mm_slabs(x3, g3_Wp, 8)
    ew3_2d = _tile_edges(ami_dis, NC * NS, ept3) + 0.0 * c2[0, 0]
    slab_k = _make_edge_kernel(n3, nap3, 128, Et3, ept3, "gcn", True,
                               "split", 8)
    parts8 = slab_k(*t8, dis3, ew3_2d, s3_2d, d3_2d, zblk)
    ami = _ami_finalize(parts8, t8, dis3, jnp.pad(g3_b, (0, 1)), ami_dis_li, n3)

    h4 = _mm(ami, g4_W)
    g4_k = _make_edge_kernel(n3, nap3, 128, Et3, ept3, "gcn", True,
                             "split", 1)
    p4 = g4_k(h4, dis3, ew3_2d, s3_2d, d3_2d, zblk)
    h4f = _gcn_finalize(p4, None, h4, dis3, g4_b, 0, n3)
    p3 = _pool(h4f, batch3.astype(jnp.int32), B)

    return _head(p1, p2, p3, fc1_W, fc1_b, fc2_W, fc2_b, out_W, out_b)
